# baseline (speedup 1.0000x reference)
import functools

import jax
import jax.numpy as jnp
from jax import lax
from jax.experimental import pallas as pl
from jax.experimental.pallas import tpu as pltpu
from jax.experimental.pallas import tpu_sc as plsc

N_NODES = 10000
D = 128
NP = 10240
DUMMY = N_NODES
E = 320000
NW = 32
K = 80
C = 128
E_PAD = NW * K * C
RPS = NP // 16
KT = 160
K0 = 112
PH = 16

_MESH = plsc.VectorSubcoreMesh(core_axis_name="c", subcore_axis_name="s")


@functools.partial(
    pl.kernel,
    mesh=_MESH,
    out_type=jax.ShapeDtypeStruct((2, NP, D), jnp.float32),
    scratch_types=[
        pltpu.VMEM((K, C), jnp.int32),
        pltpu.VMEM((C, D), jnp.float32),
        pltpu.VMEM_SHARED((NP, D), jnp.float32),
    ],
)
def _sc_degree(dst_hbm, ones_hbm, z_hbm, out_hbm, dst_v, ones_v, deg_sh):
    cid = lax.axis_index("c")
    sid = lax.axis_index("s")
    wid = sid * 2 + cid
    pltpu.sync_copy(z_hbm, deg_sh.at[pl.ds(sid * RPS, RPS)])
    pltpu.sync_copy(ones_hbm, ones_v)
    pltpu.sync_copy(dst_hbm.at[wid], dst_v)
    plsc.subcore_barrier()

    def body(j, carry):
        pltpu.sync_copy(ones_v, deg_sh.at[dst_v.at[j]], add=True)
        return carry

    lax.fori_loop(0, K, body, 0)
    plsc.subcore_barrier()
    pltpu.sync_copy(deg_sh.at[pl.ds(sid * RPS, RPS)],
                    out_hbm.at[cid, pl.ds(sid * RPS, RPS)])


@functools.partial(
    pl.kernel,
    mesh=_MESH,
    out_type=jax.ShapeDtypeStruct((2, NP, D), jnp.float32),
    scratch_types=[
        pltpu.VMEM((PH, C), jnp.int32),
        pltpu.VMEM((PH, C), jnp.int32),
        pltpu.VMEM((C, D), jnp.float32),
        pltpu.VMEM((C, D), jnp.float32),
        pltpu.VMEM_SHARED((NP, D), jnp.float32),
        pltpu.SemaphoreType.DMA,
        pltpu.SemaphoreType.DMA,
    ],
)
def _sc_seg_sum(y_hbm, src_hbm, dst_hbm, z_hbm, out_hbm,
                src_v, dst_v, buf_a, buf_b, acc_sh, sem_a, sem_b):
    cid = lax.axis_index("c")
    sid = lax.axis_index("s")
    pltpu.sync_copy(z_hbm, acc_sh.at[pl.ds(sid * RPS, RPS)])
    plsc.subcore_barrier()

    base = lax.select(cid == 0, 0, K0)
    nch = lax.select(cid == 0, K0, KT - K0)
    nph = nch // PH
    for p in range(KT // PH):
        @pl.when(p < nph)
        def _():
            pltpu.sync_copy(src_hbm.at[sid, pl.ds(base + p * PH, PH)], src_v)
            pltpu.sync_copy(dst_hbm.at[sid, pl.ds(base + p * PH, PH)], dst_v)
            pltpu.async_copy(y_hbm.at[src_v.at[0]], buf_a, sem_a)

            def body(g, carry):
                ja = 2 * g
                jb = 2 * g + 1
                pltpu.make_async_copy(y_hbm.at[src_v.at[ja]], buf_a,
                                      sem_a).wait()
                pltpu.async_copy(y_hbm.at[src_v.at[jb]], buf_b, sem_b)
                pltpu.sync_copy(buf_a, acc_sh.at[dst_v.at[ja]], add=True)
                pltpu.make_async_copy(y_hbm.at[src_v.at[jb]], buf_b,
                                      sem_b).wait()

                @pl.when(g < PH // 2 - 1)
                def _():
                    pltpu.async_copy(y_hbm.at[src_v.at[ja + 2]], buf_a, sem_a)

                pltpu.sync_copy(buf_b, acc_sh.at[dst_v.at[jb]], add=True)
                return carry

            lax.fori_loop(0, PH // 2, body, 0)
    plsc.subcore_barrier()
    pltpu.sync_copy(acc_sh.at[pl.ds(sid * RPS, RPS)],
                    out_hbm.at[cid, pl.ds(sid * RPS, RPS)])


BLK = 1024


def _dinv_from_parts(dp):
    deg = dp[0, :, 0:1] + dp[1, :, 0:1] + 1.0
    return lax.rsqrt(deg)


def _pre_body(x_ref, w_ref, dp_ref, y_ref):
    dinv = _dinv_from_parts(dp_ref[...])
    y_ref[...] = dinv * jnp.dot(x_ref[...], w_ref[...],
                                preferred_element_type=jnp.float32)


def _mid_body(acc_ref, y_ref, dp_ref, b_ref, w_ref, o_ref):
    dinv = _dinv_from_parts(dp_ref[...])
    h = jnp.maximum(dinv * (acc_ref[0] + acc_ref[1] + y_ref[...]) + b_ref[...],
                    0.0)
    o_ref[...] = dinv * jnp.dot(h, w_ref[...],
                                preferred_element_type=jnp.float32)


def _fin_body(acc_ref, y_ref, dp_ref, b_ref, o_ref):
    dinv = _dinv_from_parts(dp_ref[...])
    o_ref[...] = jnp.maximum(
        dinv * (acc_ref[0] + acc_ref[1] + y_ref[...]) + b_ref[...], 0.0)


_row_spec = pl.BlockSpec((BLK, D), lambda i: (i, 0))
_w_spec = pl.BlockSpec((D, D), lambda i: (0, 0))
_dp_spec = pl.BlockSpec((2, BLK, D), lambda i: (0, i, 0))
_acc_spec = pl.BlockSpec((2, BLK, D), lambda i: (0, i, 0))
_b_spec = pl.BlockSpec((1, D), lambda i: (0, 0))
_out_row = jax.ShapeDtypeStruct((NP, D), jnp.float32)

_tc_pre = pl.pallas_call(
    _pre_body, grid=(NP // BLK,),
    in_specs=[_row_spec, _w_spec, _dp_spec],
    out_specs=_row_spec, out_shape=_out_row)

_tc_mid = pl.pallas_call(
    _mid_body, grid=(NP // BLK,),
    in_specs=[_acc_spec, _row_spec, _dp_spec, _b_spec, _w_spec],
    out_specs=_row_spec, out_shape=_out_row)

_tc_fin = pl.pallas_call(
    _fin_body, grid=(NP // BLK,),
    in_specs=[_acc_spec, _row_spec, _dp_spec, _b_spec],
    out_specs=_row_spec, out_shape=_out_row)


@jax.jit
def kernel(node_fts, edge_index, W1, b1, W2, b2):
    src = edge_index[0].astype(jnp.int32)
    dst = edge_index[1].astype(jnp.int32)
    pad = jnp.full((E_PAD - E,), DUMMY, jnp.int32)
    srcp = jnp.concatenate([src, pad])
    dstp = jnp.concatenate([dst, pad])
    dst3 = dstp.reshape(NW, K, C)
    srcs = srcp.reshape(16, KT, C)
    dsts = dstp.reshape(16, KT, C)

    x = jnp.zeros((NP, D), jnp.float32).at[:N_NODES].set(
        node_fts.astype(jnp.float32))
    z128 = jnp.zeros((RPS, D), jnp.float32)
    ones128 = jnp.ones((C, D), jnp.float32)
    b1r = b1.reshape(1, D).astype(jnp.float32)
    b2r = b2.reshape(1, D).astype(jnp.float32)

    dp = _sc_degree(dst3, ones128, z128)
    y1 = _tc_pre(x, W1, dp)
    a1 = _sc_seg_sum(y1, srcs, dsts, z128)
    y2 = _tc_mid(a1, y1, dp, b1r, W2)
    a2 = _sc_seg_sum(y2, srcs, dsts, z128)
    out = _tc_fin(a2, y2, dp, b2r)
    return out[:N_NODES]


# speedup vs baseline: 1.0965x; 1.0965x over previous
import functools

import jax
import jax.numpy as jnp
from jax import lax
from jax.experimental import pallas as pl
from jax.experimental.pallas import tpu as pltpu
from jax.experimental.pallas import tpu_sc as plsc

N_NODES = 10000
D = 128
NP = 10240
DUMMY = N_NODES
E = 320000
NW = 32
K = 80
C = 128
E_PAD = NW * K * C
RPS = NP // 16
CS = 64
KT = E_PAD // 16 // CS
K0 = 240
PH = 40

_MESH = plsc.VectorSubcoreMesh(core_axis_name="c", subcore_axis_name="s")


@functools.partial(
    pl.kernel,
    mesh=_MESH,
    out_type=jax.ShapeDtypeStruct((2, NP, D), jnp.float32),
    scratch_types=[
        pltpu.VMEM((K, C), jnp.int32),
        pltpu.VMEM((C, D), jnp.float32),
        pltpu.VMEM_SHARED((NP, D), jnp.float32),
    ],
)
def _sc_degree(dst_hbm, ones_hbm, z_hbm, out_hbm, dst_v, ones_v, deg_sh):
    cid = lax.axis_index("c")
    sid = lax.axis_index("s")
    wid = sid * 2 + cid
    pltpu.sync_copy(z_hbm, deg_sh.at[pl.ds(sid * RPS, RPS)])
    pltpu.sync_copy(ones_hbm, ones_v)
    pltpu.sync_copy(dst_hbm.at[wid], dst_v)
    plsc.subcore_barrier()

    def body(j, carry):
        pltpu.sync_copy(ones_v, deg_sh.at[dst_v.at[j]], add=True)
        return carry

    lax.fori_loop(0, K, body, 0)
    plsc.subcore_barrier()
    pltpu.sync_copy(deg_sh.at[pl.ds(sid * RPS, RPS)],
                    out_hbm.at[cid, pl.ds(sid * RPS, RPS)])


@functools.partial(
    pl.kernel,
    mesh=_MESH,
    out_type=jax.ShapeDtypeStruct((2, NP, D), jnp.float32),
    scratch_types=[
        pltpu.VMEM((PH, CS), jnp.int32),
        pltpu.VMEM((PH, CS), jnp.int32),
        pltpu.VMEM((CS, D), jnp.float32),
        pltpu.VMEM((CS, D), jnp.float32),
        pltpu.VMEM((CS, D), jnp.float32),
        pltpu.VMEM((CS, D), jnp.float32),
        pltpu.VMEM_SHARED((NP, D), jnp.float32),
        pltpu.SemaphoreType.DMA,
        pltpu.SemaphoreType.DMA,
        pltpu.SemaphoreType.DMA,
        pltpu.SemaphoreType.DMA,
    ],
)
def _sc_seg_sum(y_hbm, src_hbm, dst_hbm, z_hbm, out_hbm,
                src_v, dst_v, b0, b1, b2, b3, acc_sh, s0, s1, s2, s3):
    cid = lax.axis_index("c")
    sid = lax.axis_index("s")
    bufs = (b0, b1, b2, b3)
    sems = (s0, s1, s2, s3)
    pltpu.sync_copy(z_hbm, acc_sh.at[pl.ds(sid * RPS, RPS)])
    plsc.subcore_barrier()

    base = lax.select(cid == 0, 0, K0)
    nch = lax.select(cid == 0, K0, KT - K0)
    nph = nch // PH
    for p in range(KT // PH):
        @pl.when(p < nph)
        def _():
            pltpu.sync_copy(src_hbm.at[sid, pl.ds(base + p * PH, PH)], src_v)
            pltpu.sync_copy(dst_hbm.at[sid, pl.ds(base + p * PH, PH)], dst_v)
            for j in range(3):
                pltpu.async_copy(y_hbm.at[src_v.at[j]], bufs[j], sems[j])

            def body(m, carry):
                for off in range(4):
                    j = 4 * m + off
                    buf = bufs[off]
                    sem = sems[off]
                    pltpu.make_async_copy(y_hbm.at[src_v.at[j]], buf,
                                          sem).wait()

                    @pl.when(j + 3 < PH)
                    def _():
                        pltpu.async_copy(y_hbm.at[src_v.at[j + 3]],
                                         bufs[(off + 3) % 4],
                                         sems[(off + 3) % 4])

                    pltpu.sync_copy(buf, acc_sh.at[dst_v.at[j]], add=True)
                return carry

            lax.fori_loop(0, PH // 4, body, 0)
    plsc.subcore_barrier()
    pltpu.sync_copy(acc_sh.at[pl.ds(sid * RPS, RPS)],
                    out_hbm.at[cid, pl.ds(sid * RPS, RPS)])


BLK = 1024


def _dinv_from_parts(dp):
    deg = dp[0, :, 0:1] + dp[1, :, 0:1] + 1.0
    return lax.rsqrt(deg)


def _pre_body(x_ref, w_ref, dp_ref, y_ref):
    dinv = _dinv_from_parts(dp_ref[...])
    y_ref[...] = dinv * jnp.dot(x_ref[...], w_ref[...],
                                preferred_element_type=jnp.float32)


def _mid_body(acc_ref, y_ref, dp_ref, b_ref, w_ref, o_ref):
    dinv = _dinv_from_parts(dp_ref[...])
    h = jnp.maximum(dinv * (acc_ref[0] + acc_ref[1] + y_ref[...]) + b_ref[...],
                    0.0)
    o_ref[...] = dinv * jnp.dot(h, w_ref[...],
                                preferred_element_type=jnp.float32)


def _fin_body(acc_ref, y_ref, dp_ref, b_ref, o_ref):
    dinv = _dinv_from_parts(dp_ref[...])
    o_ref[...] = jnp.maximum(
        dinv * (acc_ref[0] + acc_ref[1] + y_ref[...]) + b_ref[...], 0.0)


_row_spec = pl.BlockSpec((BLK, D), lambda i: (i, 0))
_w_spec = pl.BlockSpec((D, D), lambda i: (0, 0))
_dp_spec = pl.BlockSpec((2, BLK, D), lambda i: (0, i, 0))
_acc_spec = pl.BlockSpec((2, BLK, D), lambda i: (0, i, 0))
_b_spec = pl.BlockSpec((1, D), lambda i: (0, 0))
_out_row = jax.ShapeDtypeStruct((NP, D), jnp.float32)

_tc_pre = pl.pallas_call(
    _pre_body, grid=(NP // BLK,),
    in_specs=[_row_spec, _w_spec, _dp_spec],
    out_specs=_row_spec, out_shape=_out_row)

_tc_mid = pl.pallas_call(
    _mid_body, grid=(NP // BLK,),
    in_specs=[_acc_spec, _row_spec, _dp_spec, _b_spec, _w_spec],
    out_specs=_row_spec, out_shape=_out_row)

_tc_fin = pl.pallas_call(
    _fin_body, grid=(NP // BLK,),
    in_specs=[_acc_spec, _row_spec, _dp_spec, _b_spec],
    out_specs=_row_spec, out_shape=_out_row)


@jax.jit
def kernel(node_fts, edge_index, W1, b1, W2, b2):
    src = edge_index[0].astype(jnp.int32)
    dst = edge_index[1].astype(jnp.int32)
    pad = jnp.full((E_PAD - E,), DUMMY, jnp.int32)
    srcp = jnp.concatenate([src, pad])
    dstp = jnp.concatenate([dst, pad])
    dst3 = dstp.reshape(NW, K, C)
    srcs = srcp.reshape(16, KT, CS)
    dsts = dstp.reshape(16, KT, CS)

    x = jnp.zeros((NP, D), jnp.float32).at[:N_NODES].set(
        node_fts.astype(jnp.float32))
    z128 = jnp.zeros((RPS, D), jnp.float32)
    ones128 = jnp.ones((C, D), jnp.float32)
    b1r = b1.reshape(1, D).astype(jnp.float32)
    b2r = b2.reshape(1, D).astype(jnp.float32)

    dp = _sc_degree(dst3, ones128, z128)
    y1 = _tc_pre(x, W1, dp)
    a1 = _sc_seg_sum(y1, srcs, dsts, z128)
    y2 = _tc_mid(a1, y1, dp, b1r, W2)
    a2 = _sc_seg_sum(y2, srcs, dsts, z128)
    out = _tc_fin(a2, y2, dp, b2r)
    return out[:N_NODES]
